# baseline (device time: 37911 ns/iter reference)
import jax
import jax.numpy as jnp
from jax import lax
from jax.experimental import pallas as pl
from jax.experimental.pallas import tpu as pltpu

N_DEV = 4
B_PER = 2
SQ = 128
SKV = 128
HQ = 16
H_PER = HQ // N_DEV
DH = 64
DM = 512
DG = H_PER * DH


def kernel(x, Wq, K_ext, V_ext, Wo):
    def body(x_ref, wq_ref, k_ref, v_ref, wo_ref, out_ref,
             wq_comm, wo_comm, q_scr, ctx_scr, send_sems, recv_sems):
        my = lax.axis_index("i")

        wq_comm[pl.ds(my, 1)] = wq_ref[...].astype(jnp.bfloat16)[None]
        wo_comm[pl.ds(my, 1)] = wo_ref[...].astype(jnp.bfloat16)[None]

        barrier = pltpu.get_barrier_semaphore()
        for o in range(1, N_DEV):
            peer = lax.rem(my + o, N_DEV)
            pl.semaphore_signal(barrier, inc=1, device_id=(peer,),
                                device_id_type=pl.DeviceIdType.MESH)
        pl.semaphore_wait(barrier, N_DEV - 1)

        sends = []
        for o in range(1, N_DEV):
            tgt = lax.rem(my + o, N_DEV)
            for j, comm in enumerate((wq_comm, wo_comm)):
                idx = (o - 1) + 3 * j
                rdma = pltpu.make_async_remote_copy(
                    src_ref=comm.at[my],
                    dst_ref=comm.at[my],
                    send_sem=send_sems.at[idx],
                    recv_sem=recv_sems.at[idx],
                    device_id=(tgt,),
                    device_id_type=pl.DeviceIdType.MESH,
                )
                rdma.start()
                sends.append(rdma)

        for o in range(1, N_DEV):
            src = lax.rem(my - o + N_DEV, N_DEV)
            for j, comm in enumerate((wq_comm, wo_comm)):
                idx = (o - 1) + 3 * j
                rdma = pltpu.make_async_remote_copy(
                    src_ref=comm.at[src],
                    dst_ref=comm.at[src],
                    send_sem=send_sems.at[idx],
                    recv_sem=recv_sems.at[idx],
                    device_id=(src,),
                    device_id_type=pl.DeviceIdType.MESH,
                )
                rdma.wait_recv()

        xb = x_ref[...].reshape(B_PER * SQ, DM).astype(jnp.bfloat16)

        qb_blk = lax.broadcasted_iota(jnp.int32, (SQ, SKV), 0) // 64
        kb_blk = lax.broadcasted_iota(jnp.int32, (SQ, SKV), 1) // 64
        mask = (qb_blk == kb_blk) | ((kb_blk % 4) == (qb_blk % 4))
        neg = jnp.float32(-1e9)

        acc = jnp.zeros((B_PER * SQ, DM), jnp.float32)
        for g in range(N_DEV):
            q = jnp.dot(xb, wq_comm[g], preferred_element_type=jnp.float32)
            q_scr[...] = (q * 0.125).astype(jnp.bfloat16)
            for b in range(B_PER):
                bidx = my * B_PER + b
                for hl in range(H_PER):
                    hd = g * H_PER + hl
                    qbh = q_scr[b * SQ:(b + 1) * SQ, hl * DH:(hl + 1) * DH]
                    kbh = k_ref[pl.ds(bidx, 1), :, pl.ds(hd, 1), :].reshape(
                        SKV, DH).astype(jnp.bfloat16)
                    s = lax.dot_general(
                        qbh, kbh, (((1,), (1,)), ((), ())),
                        preferred_element_type=jnp.float32)
                    s = jnp.where(mask, s, neg)
                    s = s - jnp.max(s, axis=-1, keepdims=True)
                    w = jnp.exp(s)
                    w = w / jnp.sum(w, axis=-1, keepdims=True)
                    vbh = v_ref[pl.ds(bidx, 1), :, pl.ds(hd, 1), :].reshape(
                        SKV, DH).astype(jnp.bfloat16)
                    c = jnp.dot(w.astype(jnp.bfloat16), vbh,
                                preferred_element_type=jnp.float32)
                    ctx_scr[b * SQ:(b + 1) * SQ,
                            hl * DH:(hl + 1) * DH] = c.astype(jnp.bfloat16)
            acc = acc + jnp.dot(ctx_scr[...], wo_comm[g],
                                preferred_element_type=jnp.float32)
        out_ref[...] = acc.reshape(B_PER, SQ, DM)

        for r in sends:
            r.wait_send()

    out_shape = jax.ShapeDtypeStruct((B_PER, SQ, DM), jnp.float32)
    return pl.pallas_call(
        body,
        out_shape=out_shape,
        in_specs=[pl.BlockSpec(memory_space=pltpu.VMEM)] * 5,
        out_specs=pl.BlockSpec(memory_space=pltpu.VMEM),
        scratch_shapes=[
            pltpu.VMEM((N_DEV, DM, DG), jnp.bfloat16),
            pltpu.VMEM((N_DEV, DG, DM), jnp.bfloat16),
            pltpu.VMEM((B_PER * SQ, DG), jnp.bfloat16),
            pltpu.VMEM((B_PER * SQ, DG), jnp.bfloat16),
            pltpu.SemaphoreType.DMA((6,)),
            pltpu.SemaphoreType.DMA((6,)),
        ],
        compiler_params=pltpu.CompilerParams(collective_id=0),
    )(x, Wq, K_ext, V_ext, Wo)


# device time: 34255 ns/iter; 1.1067x vs baseline; 1.1067x over previous
import jax
import jax.numpy as jnp
from jax import lax
from jax.experimental import pallas as pl
from jax.experimental.pallas import tpu as pltpu

N_DEV = 4
B_PER = 2
SQ = 128
SKV = 128
HQ = 16
H_PER = HQ // N_DEV
DH = 64
DM = 512
DG = H_PER * DH


def kernel(x, Wq, K_ext, V_ext, Wo):
    def body(x_ref, wq_ref, k_ref, v_ref, wo_ref, out_ref,
             wq_comm, wo_comm, k_scr, v_scr, q_scr, ctx_scr,
             send_sems, recv_sems, kv_sems):
        my = lax.axis_index("i")

        k_dma = pltpu.make_async_copy(
            k_ref.at[pl.ds(my * B_PER, B_PER)], k_scr, kv_sems.at[0])
        v_dma = pltpu.make_async_copy(
            v_ref.at[pl.ds(my * B_PER, B_PER)], v_scr, kv_sems.at[1])
        k_dma.start()
        v_dma.start()

        wq_comm[pl.ds(my, 1)] = wq_ref[...].astype(jnp.bfloat16)[None]
        wo_comm[pl.ds(my, 1)] = wo_ref[...].astype(jnp.bfloat16)[None]

        barrier = pltpu.get_barrier_semaphore()
        for o in range(1, N_DEV):
            peer = lax.rem(my + o, N_DEV)
            pl.semaphore_signal(barrier, inc=1, device_id=(peer,),
                                device_id_type=pl.DeviceIdType.MESH)
        pl.semaphore_wait(barrier, N_DEV - 1)

        sends = []
        for o in range(1, N_DEV):
            tgt = lax.rem(my + o, N_DEV)
            for j, comm in enumerate((wq_comm, wo_comm)):
                idx = (o - 1) + 3 * j
                rdma = pltpu.make_async_remote_copy(
                    src_ref=comm.at[my],
                    dst_ref=comm.at[my],
                    send_sem=send_sems.at[idx],
                    recv_sem=recv_sems.at[idx],
                    device_id=(tgt,),
                    device_id_type=pl.DeviceIdType.MESH,
                )
                rdma.start()
                sends.append(rdma)

        xb = x_ref[...].reshape(B_PER * SQ, DM).astype(jnp.bfloat16)
        qb_blk = lax.broadcasted_iota(jnp.int32, (SQ, SKV), 0) // 64
        kb_blk = lax.broadcasted_iota(jnp.int32, (SQ, SKV), 1) // 64
        keep = (qb_blk == kb_blk) | ((kb_blk % 4) == (qb_blk % 4))
        mask_bias = jnp.where(keep, jnp.float32(0.0), jnp.float32(-1e9))
        k_dma.wait()
        v_dma.wait()

        for o in range(1, N_DEV):
            src = lax.rem(my - o + N_DEV, N_DEV)
            for j, comm in enumerate((wq_comm, wo_comm)):
                idx = (o - 1) + 3 * j
                rdma = pltpu.make_async_remote_copy(
                    src_ref=comm.at[src],
                    dst_ref=comm.at[src],
                    send_sem=send_sems.at[idx],
                    recv_sem=recv_sems.at[idx],
                    device_id=(src,),
                    device_id_type=pl.DeviceIdType.MESH,
                )
                rdma.wait_recv()

        acc = jnp.zeros((B_PER * SQ, DM), jnp.float32)
        for g in range(N_DEV):
            q = jnp.dot(xb, wq_comm[g], preferred_element_type=jnp.float32)
            q_scr[...] = (q * 0.125).astype(jnp.bfloat16)
            for b in range(B_PER):
                for hl in range(H_PER):
                    hd = g * H_PER + hl
                    qbh = q_scr[b * SQ:(b + 1) * SQ, hl * DH:(hl + 1) * DH]
                    kbh = k_scr[b, :, hd, :].astype(jnp.bfloat16)
                    s = lax.dot_general(
                        qbh, kbh, (((1,), (1,)), ((), ())),
                        preferred_element_type=jnp.float32)
                    w = jnp.exp(s + mask_bias)
                    inv = 1.0 / jnp.sum(w, axis=-1, keepdims=True)
                    vbh = v_scr[b, :, hd, :].astype(jnp.bfloat16)
                    c = jnp.dot(w.astype(jnp.bfloat16), vbh,
                                preferred_element_type=jnp.float32) * inv
                    ctx_scr[b * SQ:(b + 1) * SQ,
                            hl * DH:(hl + 1) * DH] = c.astype(jnp.bfloat16)
            acc = acc + jnp.dot(ctx_scr[...], wo_comm[g],
                                preferred_element_type=jnp.float32)
        out_ref[...] = acc.reshape(B_PER, SQ, DM)

        for r in sends:
            r.wait_send()

    out_shape = jax.ShapeDtypeStruct((B_PER, SQ, DM), jnp.float32)
    vmem = pl.BlockSpec(memory_space=pltpu.VMEM)
    anym = pl.BlockSpec(memory_space=pl.ANY)
    return pl.pallas_call(
        body,
        out_shape=out_shape,
        in_specs=[vmem, vmem, anym, anym, vmem],
        out_specs=vmem,
        scratch_shapes=[
            pltpu.VMEM((N_DEV, DM, DG), jnp.bfloat16),
            pltpu.VMEM((N_DEV, DG, DM), jnp.bfloat16),
            pltpu.VMEM((B_PER, SKV, HQ, DH), jnp.float32),
            pltpu.VMEM((B_PER, SKV, HQ, DH), jnp.float32),
            pltpu.VMEM((B_PER * SQ, DG), jnp.bfloat16),
            pltpu.VMEM((B_PER * SQ, DG), jnp.bfloat16),
            pltpu.SemaphoreType.DMA((6,)),
            pltpu.SemaphoreType.DMA((6,)),
            pltpu.SemaphoreType.DMA((2,)),
        ],
        compiler_params=pltpu.CompilerParams(collective_id=0),
    )(x, Wq, K_ext, V_ext, Wo)


# device time: 18131 ns/iter; 2.0909x vs baseline; 1.8893x over previous
import jax
import jax.numpy as jnp
from jax import lax
from jax.experimental import pallas as pl
from jax.experimental.pallas import tpu as pltpu

N_DEV = 4
B_PER = 2
SQ = 128
SKV = 128
HQ = 16
H_PER = HQ // N_DEV
DH = 64
DM = 512
DG = H_PER * DH

DO_COMM = True


def kernel(x, Wq, K_ext, V_ext, Wo):
    def body(x_ref, wq_ref, k_ref, v_ref, wo_ref, out_ref,
             wq_comm, wo_comm, k_scr, v_scr, q_scr, ctx_scr,
             send_sems, recv_sems, kv_sems):
        my = lax.axis_index("i")

        k_dma = pltpu.make_async_copy(
            k_ref.at[pl.ds(my * B_PER, B_PER)], k_scr, kv_sems.at[0])
        v_dma = pltpu.make_async_copy(
            v_ref.at[pl.ds(my * B_PER, B_PER)], v_scr, kv_sems.at[1])
        k_dma.start()
        v_dma.start()

        wq_comm[pl.ds(my, 1)] = wq_ref[...].astype(jnp.bfloat16)[None]
        wo_comm[pl.ds(my, 1)] = wo_ref[...].astype(jnp.bfloat16)[None]

        barrier = pltpu.get_barrier_semaphore() if DO_COMM else None
        for o in range(1, N_DEV) if DO_COMM else []:
            peer = lax.rem(my + o, N_DEV)
            pl.semaphore_signal(barrier, inc=1, device_id=(peer,),
                                device_id_type=pl.DeviceIdType.MESH)
        if DO_COMM:
            pl.semaphore_wait(barrier, N_DEV - 1)

        sends = []
        for o in range(1, N_DEV) if DO_COMM else []:
            tgt = lax.rem(my + o, N_DEV)
            for j, comm in enumerate((wq_comm, wo_comm)):
                idx = (o - 1) + 3 * j
                rdma = pltpu.make_async_remote_copy(
                    src_ref=comm.at[my],
                    dst_ref=comm.at[my],
                    send_sem=send_sems.at[idx],
                    recv_sem=recv_sems.at[idx],
                    device_id=(tgt,),
                    device_id_type=pl.DeviceIdType.MESH,
                )
                rdma.start()
                sends.append(rdma)

        xb = x_ref[...].reshape(B_PER * SQ, DM).astype(jnp.bfloat16)
        qb_blk = lax.broadcasted_iota(jnp.int32, (SQ, SKV), 0) // 64
        kb_blk = lax.broadcasted_iota(jnp.int32, (SQ, SKV), 1) // 64
        keep = (qb_blk == kb_blk) | ((kb_blk % 4) == (qb_blk % 4))
        mask_bias = jnp.where(keep, jnp.float32(0.0), jnp.float32(-1e9))
        k_dma.wait()
        v_dma.wait()

        for o in range(1, N_DEV) if DO_COMM else []:
            src = lax.rem(my - o + N_DEV, N_DEV)
            for j, comm in enumerate((wq_comm, wo_comm)):
                idx = (o - 1) + 3 * j
                rdma = pltpu.make_async_remote_copy(
                    src_ref=comm.at[src],
                    dst_ref=comm.at[src],
                    send_sem=send_sems.at[idx],
                    recv_sem=recv_sems.at[idx],
                    device_id=(src,),
                    device_id_type=pl.DeviceIdType.MESH,
                )
                rdma.wait_recv()

        acc = jnp.zeros((B_PER * SQ, DM), jnp.float32)
        for g in range(N_DEV):
            q = jnp.dot(xb, wq_comm[g], preferred_element_type=jnp.float32)
            q_scr[...] = (q * 0.125).astype(jnp.bfloat16)
            for b in range(B_PER):
                for hl in range(H_PER):
                    hd = g * H_PER + hl
                    qbh = q_scr[b * SQ:(b + 1) * SQ, hl * DH:(hl + 1) * DH]
                    kbh = k_scr[b, :, hd, :].astype(jnp.bfloat16)
                    s = lax.dot_general(
                        qbh, kbh, (((1,), (1,)), ((), ())),
                        preferred_element_type=jnp.float32)
                    w = jnp.exp(s + mask_bias)
                    inv = 1.0 / jnp.sum(w, axis=-1, keepdims=True)
                    vbh = v_scr[b, :, hd, :].astype(jnp.bfloat16)
                    c = jnp.dot(w.astype(jnp.bfloat16), vbh,
                                preferred_element_type=jnp.float32) * inv
                    ctx_scr[b * SQ:(b + 1) * SQ,
                            hl * DH:(hl + 1) * DH] = c.astype(jnp.bfloat16)
            acc = acc + jnp.dot(ctx_scr[...], wo_comm[g],
                                preferred_element_type=jnp.float32)
        out_ref[...] = acc.reshape(B_PER, SQ, DM)

        for r in sends:
            r.wait_send()

    out_shape = jax.ShapeDtypeStruct((B_PER, SQ, DM), jnp.float32)
    vmem = pl.BlockSpec(memory_space=pltpu.VMEM)
    anym = pl.BlockSpec(memory_space=pl.ANY)
    return pl.pallas_call(
        body,
        out_shape=out_shape,
        in_specs=[vmem, vmem, anym, anym, vmem],
        out_specs=vmem,
        scratch_shapes=[
            pltpu.VMEM((N_DEV, DM, DG), jnp.bfloat16),
            pltpu.VMEM((N_DEV, DG, DM), jnp.bfloat16),
            pltpu.VMEM((B_PER, SKV, HQ, DH), jnp.float32),
            pltpu.VMEM((B_PER, SKV, HQ, DH), jnp.float32),
            pltpu.VMEM((B_PER * SQ, DG), jnp.bfloat16),
            pltpu.VMEM((B_PER * SQ, DG), jnp.bfloat16),
            pltpu.SemaphoreType.DMA((6,)),
            pltpu.SemaphoreType.DMA((6,)),
            pltpu.SemaphoreType.DMA((2,)),
        ],
        compiler_params=pltpu.CompilerParams(
            collective_id=0 if DO_COMM else None),
    )(x, Wq, K_ext, V_ext, Wo)
